# baseline (speedup 1.0000x reference)
import functools

import numpy as np
import jax
import jax.numpy as jnp
from jax.experimental import pallas as pl

_B, _C, _W0, _H0 = 8, 192, 56, 56
_HEADS, _HD = 4, 24
_OC = _HEADS * _HD
_S = _W0 * _H0
_NJ = 16
_NJB = _HEADS * _NJ
_NPTS = _B * _HEADS * _S
_NORTH = _B * _HEADS * 4 * 16


def _constants():
    s = np.arange(_S)
    w, h = s // _H0, s % _H0
    j = (w // 28) * 8 + (h // 28) * 4 + ((w // 14) % 2) * 2 + ((h // 14) % 2)
    rows = np.arange(_NJ)[:, None]
    pool = (rows == j[None, :]).astype(np.float32) / 196.0
    validb = np.tile(((rows // 4) == (j[None, :] // 4)), (_HEADS, 1)
                     ).astype(np.float32)
    ridx64 = (np.arange(_NJB, dtype=np.float32) % _NJ)[:, None]
    jb = np.arange(_NJB)
    cg = np.arange(_OC)
    bd_jc = ((jb[:, None] // _NJ) == (cg[None, :] // _HD)).astype(np.float32)
    eye64 = np.eye(_NJB, dtype=np.float32)
    ccm = (((jb[:, None] // _NJ) == (jb[None, :] // _NJ)) &
           ((jb[:, None] % _NJ) // 4 == (jb[None, :] % _NJ) // 4)
           ).astype(np.float32)
    return pool, validb, ridx64, bd_jc, eye64, ccm


_POOL, _VALIDB, _RIDX64, _BDJC, _EYE64, _CCM = _constants()


def _fec_kernel(x_ref, wfv_ref, bfv_ref, wp_ref, bp_ref,
                ab_ref, pool_ref, valid_ref, ridx_ref,
                bdjc_ref, eye64_ref, ccm_ref,
                out_ref, lc_ref, ls_ref, lo_ref):
    i = pl.program_id(0)

    @pl.when(i == 0)
    def _init():
        lc_ref[...] = jnp.zeros((1, 1), jnp.float32)
        ls_ref[...] = jnp.zeros((1, 1), jnp.float32)
        lo_ref[...] = jnp.zeros((1, 1), jnp.float32)

    f32 = jnp.float32
    bf16 = jnp.bfloat16
    dot = functools.partial(jax.lax.dot_general, preferred_element_type=f32)
    hdot = functools.partial(jax.lax.dot_general, preferred_element_type=f32,
                             precision=jax.lax.Precision.HIGHEST)
    xb = x_ref[0].astype(bf16)
    fv = dot(wfv_ref[...].astype(bf16), xb,
             (((1,), (0,)), ((), ()))) + bfv_ref[...]
    feat, val = fv[:_OC], fv[_OC:]
    cents_t = hdot(pool_ref[...], fv, (((1,), (1,)), ((), ())))
    cent_t, vcent_t = cents_t[:, :_OC], cents_t[:, _OC:]
    alpha = ab_ref[0, 0]
    beta = ab_ref[0, 1]
    eps = 1e-12

    f2 = (feat * feat).reshape(_HEADS, _HD, _S)
    pn = jnp.sqrt(jnp.sum(f2, axis=1, keepdims=True)) + eps
    fhat = (feat.reshape(_HEADS, _HD, _S) /
            pn).reshape(_OC, _S).astype(bf16)
    chat_parts = []
    for e in range(_HEADS):
        ce = cent_t[:, e * _HD:(e + 1) * _HD]
        cn = jnp.sqrt(jnp.sum(ce * ce, axis=1, keepdims=True)) + eps
        chat_parts.append(ce / cn)
    centhat_t = jnp.concatenate(chat_parts, axis=1)
    bd = (jnp.tile(centhat_t, (_HEADS, 1)) * bdjc_ref[...]).astype(bf16)
    cos_all = dot(bd, fhat, (((1,), (0,)), ((), ())))

    sim = jax.nn.sigmoid(beta + alpha * cos_all)
    simv = jnp.where(valid_ref[...] > 0.0, sim, -1.0)
    sim4 = simv.reshape(_HEADS, _NJ, _S)
    smax4 = jnp.max(sim4, axis=1, keepdims=True)
    ridx = ridx_ref[...]
    smax_b = jnp.broadcast_to(smax4, (_HEADS, _NJ, _S)).reshape(_NJB, _S)
    cand = jnp.where(simv == smax_b, ridx, 1e9)
    idx4 = jnp.min(cand.reshape(_HEADS, _NJ, _S), axis=1, keepdims=True)
    idx_b = jnp.broadcast_to(idx4, (_HEADS, _NJ, _S)).reshape(_NJB, _S)
    onehot = (ridx == idx_b).astype(f32)
    weight_bd = onehot * smax_b
    smax2 = jnp.max(jnp.where(onehot > 0.0, -1.0, simv
                              ).reshape(_HEADS, _NJ, _S), axis=1)
    lc_acc = jnp.sum(smax4)
    ls_acc = jnp.sum(smax2)

    agg_t = hdot(weight_bd, val, (((1,), (1,)), ((), ())))
    cnt = jnp.sum(onehot, axis=1, keepdims=True)
    vce_t = jnp.tile(vcent_t, (_HEADS, 1))
    outc_t = ((agg_t + vce_t) / (cnt + 1.0)) * bdjc_ref[...]
    disp = hdot(outc_t, weight_bd, (((0,), (0,)), ((), ())))
    cc_all = dot(bd, bd, (((1,), (1,)), ((), ())))
    lo_acc = jnp.sum(((cc_all - eye64_ref[...]) ** 2) * ccm_ref[...])

    out = dot(wp_ref[...].astype(bf16), disp.astype(bf16),
              (((1,), (0,)), ((), ()))) + bp_ref[...]
    out_ref[0] = out
    lc_ref[...] = lc_ref[...] + lc_acc
    ls_ref[...] = ls_ref[...] + ls_acc
    lo_ref[...] = lo_ref[...] + lo_acc

    @pl.when(i == _B - 1)
    def _finalize():
        lc_ref[...] = -lc_ref[...] / _NPTS
        ls_ref[...] = ls_ref[...] / _NPTS
        lo_ref[...] = lo_ref[...] / _NORTH


def kernel(x, Wf, bf, Wv, bv, Wp, bp, sim_alpha, sim_beta):
    f32 = jnp.float32
    xf = x.reshape(_B, _C, _S)
    wfv = jnp.concatenate([Wf, Wv], axis=0)
    bfv = jnp.concatenate([bf, bv]).reshape(2 * _OC, 1)
    ab = jnp.concatenate([sim_alpha, sim_beta]).reshape(1, 2).astype(f32)

    full = lambda shape: pl.BlockSpec(shape, lambda i: (0,) * len(shape))
    out, lc, ls, lo = pl.pallas_call(
        _fec_kernel,
        grid=(_B,),
        in_specs=[
            pl.BlockSpec((1, _C, _S), lambda i: (i, 0, 0)),
            full((2 * _OC, _C)), full((2 * _OC, 1)),
            full((_C, _OC)), full((_C, 1)),
            full((1, 2)),
            full((_NJ, _S)), full((_NJB, _S)), full((_NJB, 1)),
            full((_NJB, _OC)), full((_NJB, _NJB)), full((_NJB, _NJB)),
        ],
        out_specs=[
            pl.BlockSpec((1, _C, _S), lambda i: (i, 0, 0)),
            full((1, 1)), full((1, 1)), full((1, 1)),
        ],
        out_shape=[
            jax.ShapeDtypeStruct((_B, _C, _S), f32),
            jax.ShapeDtypeStruct((1, 1), f32),
            jax.ShapeDtypeStruct((1, 1), f32),
            jax.ShapeDtypeStruct((1, 1), f32),
        ],
    )(xf, wfv, bfv, Wp, bp.reshape(_C, 1), ab,
      jnp.asarray(_POOL), jnp.asarray(_VALIDB), jnp.asarray(_RIDX64),
      jnp.asarray(_BDJC), jnp.asarray(_EYE64), jnp.asarray(_CCM))
    return out.reshape(_B, _C, _W0, _H0), lc[0, 0], ls[0, 0], lo[0, 0]
